# Initial kernel scaffold; baseline (speedup 1.0000x reference)
import jax
import jax.numpy as jnp
from jax.experimental import pallas as pl


def kernel(x, y, Wq, bq, Wk, bk, Wv, bv, Wqy, bqy, Wky, bky, Wvy, bvy, gamma, gamma_y, gamma_cx, gamma_cy, wx1, wx2, wy1, wy2):
    raise NotImplementedError("write your pallas kernel here")


# speedup vs baseline: 2.2563x; 2.2563x over previous
import jax
import jax.numpy as jnp
from jax.experimental import pallas as pl
from jax.experimental.pallas import tpu as pltpu

EPS = 1e-6


def _dot(a, b, dims):
    return jax.lax.dot_general(a, b, (dims, ((), ())),
                               preferred_element_type=jnp.float32)


def _stats_kernel(nb1,
                  x_ref, y_ref, wq_ref, bq_ref, wk_ref, bk_ref, wv_ref, bv_ref,
                  wqy_ref, bqy_ref, wky_ref, bky_ref, wvy_ref, bvy_ref,
                  qx_ref, qy_ref, matx_ref, maty_ref,
                  ks_ref, kys_ref, vs_ref, vys_ref):
    i = pl.program_id(1)

    @pl.when(i == 0)
    def _():
        matx_ref[...] = jnp.zeros_like(matx_ref)
        maty_ref[...] = jnp.zeros_like(maty_ref)
        ks_ref[...] = jnp.zeros_like(ks_ref)
        kys_ref[...] = jnp.zeros_like(kys_ref)
        vs_ref[...] = jnp.zeros_like(vs_ref)
        vys_ref[...] = jnp.zeros_like(vys_ref)

    def stream(t_ref, wq, bq, wk, bk, wv, bv, q_out, mat_out, ks_out, vs_out):
        t = t_ref[0]
        q = _dot(t, wq[...], ((1,), (1,))) + bq[...]
        q = q / jnp.sqrt(jnp.sum(q * q, axis=1, keepdims=True))
        k = _dot(t, wk[...], ((1,), (1,))) + bk[...]
        k = k / jnp.sqrt(jnp.sum(k * k, axis=1, keepdims=True))
        v = _dot(t, wv[...], ((1,), (1,))) + bv[...]
        q_out[0] = q
        mat_out[0] += _dot(k, v, ((0,), (0,)))
        ks_out[0] += jnp.sum(k, axis=0, keepdims=True)
        vs_out[0] += jnp.sum(v, axis=0, keepdims=True)

    stream(x_ref, wq_ref, bq_ref, wk_ref, bk_ref, wv_ref, bv_ref,
           qx_ref, matx_ref, ks_ref, vs_ref)
    stream(y_ref, wqy_ref, bqy_ref, wky_ref, bky_ref, wvy_ref, bvy_ref,
           qy_ref, maty_ref, kys_ref, vys_ref)

    @pl.when(i == nb1 - 1)
    def _():
        ks_ref[...] += EPS
        kys_ref[...] += EPS


def _out_kernel(n_total,
                qx_ref, qy_ref, matx_ref, maty_ref, kp_ref,
                vs_ref, vys_ref, s_ref, fx_ref, fy_ref):
    qx = qx_ref[0]
    qy = qy_ref[0]
    tq = _dot(qx, kp_ref[0], ((1,), (0,)))
    tqy = _dot(qy, kp_ref[0], ((1,), (0,)))
    n = jnp.float32(n_total)
    ax1 = s_ref[0] / (n + tq[:, 0:1])
    ax2 = s_ref[1] / (n + tq[:, 1:2])
    ay1 = s_ref[2] / (n + tqy[:, 1:2])
    ay2 = s_ref[3] / (n + tqy[:, 0:1])
    qm_x1 = _dot(qx, matx_ref[0], ((1,), (0,)))
    qm_x2 = _dot(qx, maty_ref[0], ((1,), (0,)))
    qm_y1 = _dot(qy, maty_ref[0], ((1,), (0,)))
    qm_y2 = _dot(qy, matx_ref[0], ((1,), (0,)))
    vs = vs_ref[0]
    vys = vys_ref[0]
    fx_ref[0] = ax1 * (vs + qm_x1) + ax2 * (vs + qm_x2)
    fy_ref[0] = ay1 * (vys + qm_y1) + ay2 * (vys + qm_y2)


def _run(x, y, Wq, bq, Wk, bk, Wv, bv, Wqy, bqy, Wky, bky, Wvy, bvy,
         gamma, gamma_y, gamma_cx, gamma_cy, wx1, wx2, wy1, wy2,
         interpret=False):
    b, n, c = x.shape
    d = Wq.shape[0]
    bn1 = min(2048, n)
    bn2 = min(2048, n)
    nb1 = n // bn1
    nb2 = n // bn2

    row_spec = lambda bn: pl.BlockSpec((1, bn, c), lambda bi, i: (bi, i, 0))
    w_spec = lambda r, cc: pl.BlockSpec((r, cc), lambda bi, i: (0, 0))
    stat_spec = lambda r, cc: pl.BlockSpec((1, r, cc), lambda bi, i: (bi, 0, 0))
    f32 = jnp.float32

    import functools
    stats = pl.pallas_call(
        functools.partial(_stats_kernel, nb1),
        grid=(b, nb1),
        in_specs=[
            row_spec(bn1), row_spec(bn1),
            w_spec(d, c), w_spec(1, d), w_spec(d, c), w_spec(1, d),
            w_spec(c, c), w_spec(1, c),
            w_spec(d, c), w_spec(1, d), w_spec(d, c), w_spec(1, d),
            w_spec(c, c), w_spec(1, c),
        ],
        out_specs=[
            pl.BlockSpec((1, bn1, d), lambda bi, i: (bi, i, 0)),
            pl.BlockSpec((1, bn1, d), lambda bi, i: (bi, i, 0)),
            stat_spec(d, c), stat_spec(d, c),
            stat_spec(1, d), stat_spec(1, d),
            stat_spec(1, c), stat_spec(1, c),
        ],
        out_shape=[
            jax.ShapeDtypeStruct((b, n, d), f32),
            jax.ShapeDtypeStruct((b, n, d), f32),
            jax.ShapeDtypeStruct((b, d, c), f32),
            jax.ShapeDtypeStruct((b, d, c), f32),
            jax.ShapeDtypeStruct((b, 1, d), f32),
            jax.ShapeDtypeStruct((b, 1, d), f32),
            jax.ShapeDtypeStruct((b, 1, c), f32),
            jax.ShapeDtypeStruct((b, 1, c), f32),
        ],
        compiler_params=pltpu.CompilerParams(
            dimension_semantics=("parallel", "arbitrary")),
        name="linattn_stats",
        interpret=interpret,
    )(x, y,
      Wq, bq.reshape(1, d), Wk, bk.reshape(1, d), Wv, bv.reshape(1, c),
      Wqy, bqy.reshape(1, d), Wky, bky.reshape(1, d), Wvy, bvy.reshape(1, c))
    qx, qy, matx, maty, ks, kys, vs, vys = stats

    kp = jnp.transpose(jnp.concatenate([ks, kys], axis=1), (0, 2, 1))
    s = jnp.stack([gamma[0] * wx1, gamma_cx[0] * wx2,
                   gamma_y[0] * wy1, gamma_cy[0] * wy2]).astype(f32)

    fx, fy = pl.pallas_call(
        functools.partial(_out_kernel, n),
        grid=(b, nb2),
        in_specs=[
            pl.BlockSpec((1, bn2, d), lambda bi, i: (bi, i, 0)),
            pl.BlockSpec((1, bn2, d), lambda bi, i: (bi, i, 0)),
            stat_spec(d, c), stat_spec(d, c),
            pl.BlockSpec((1, d, 2), lambda bi, i: (bi, 0, 0)),
            stat_spec(1, c), stat_spec(1, c),
            pl.BlockSpec(memory_space=pltpu.SMEM),
        ],
        out_specs=[row_spec(bn2), row_spec(bn2)],
        out_shape=[
            jax.ShapeDtypeStruct((b, n, c), f32),
            jax.ShapeDtypeStruct((b, n, c), f32),
        ],
        compiler_params=pltpu.CompilerParams(
            dimension_semantics=("parallel", "arbitrary")),
        name="linattn_out",
        interpret=interpret,
    )(qx, qy, matx, maty, kp, vs, vys, s)
    return fx, fy


def kernel(x, y, Wq, bq, Wk, bk, Wv, bv, Wqy, bqy, Wky, bky, Wvy, bvy,
           gamma, gamma_y, gamma_cx, gamma_cy, wx1, wx2, wy1, wy2):
    return _run(x, y, Wq, bq, Wk, bk, Wv, bv, Wqy, bqy, Wky, bky, Wvy, bvy,
                gamma, gamma_y, gamma_cx, gamma_cy, wx1, wx2, wy1, wy2)


# speedup vs baseline: 2.3173x; 1.0270x over previous
import jax
import jax.numpy as jnp
from jax.experimental import pallas as pl
from jax.experimental.pallas import tpu as pltpu

EPS = 1e-6


def _dot(a, b, dims):
    return jax.lax.dot_general(a, b, (dims, ((), ())),
                               preferred_element_type=jnp.float32)


def _stats_kernel(nb1,
                  x_ref, y_ref, wq_ref, bq_ref, wk_ref, bk_ref, wv_ref, bv_ref,
                  wqy_ref, bqy_ref, wky_ref, bky_ref, wvy_ref, bvy_ref,
                  qx_ref, qy_ref, matx_ref, maty_ref,
                  ks_ref, kys_ref, vs_ref, vys_ref):
    i = pl.program_id(1)

    @pl.when(i == 0)
    def _():
        matx_ref[...] = jnp.zeros_like(matx_ref)
        maty_ref[...] = jnp.zeros_like(maty_ref)
        ks_ref[...] = jnp.zeros_like(ks_ref)
        kys_ref[...] = jnp.zeros_like(kys_ref)
        vs_ref[...] = jnp.zeros_like(vs_ref)
        vys_ref[...] = jnp.zeros_like(vys_ref)

    def stream(t_ref, wq, bq, wk, bk, wv, bv, q_out, mat_out, ks_out, vs_out):
        t = t_ref[0]
        q = _dot(t, wq[...], ((1,), (1,))) + bq[...]
        q = q * jax.lax.rsqrt(jnp.sum(q * q, axis=1, keepdims=True))
        k = _dot(t, wk[...], ((1,), (1,))) + bk[...]
        k = k * jax.lax.rsqrt(jnp.sum(k * k, axis=1, keepdims=True))
        v = _dot(t, wv[...], ((1,), (1,))) + bv[...]
        q_out[0] = q
        mat_out[0] += _dot(k, v, ((0,), (0,)))
        ks_out[0] += jnp.sum(k, axis=0, keepdims=True)
        vs_out[0] += jnp.sum(v, axis=0, keepdims=True)

    stream(x_ref, wq_ref, bq_ref, wk_ref, bk_ref, wv_ref, bv_ref,
           qx_ref, matx_ref, ks_ref, vs_ref)
    stream(y_ref, wqy_ref, bqy_ref, wky_ref, bky_ref, wvy_ref, bvy_ref,
           qy_ref, maty_ref, kys_ref, vys_ref)

    @pl.when(i == nb1 - 1)
    def _():
        ks_ref[...] += EPS
        kys_ref[...] += EPS


def _out_kernel(n_total,
                qx_ref, qy_ref, matx_ref, maty_ref, kp_ref,
                vs_ref, vys_ref, s_ref, fx_ref, fy_ref):
    qx = qx_ref[0]
    qy = qy_ref[0]
    tq = _dot(qx, kp_ref[0], ((1,), (0,)))
    tqy = _dot(qy, kp_ref[0], ((1,), (0,)))
    n = jnp.float32(n_total)
    ax1 = s_ref[0] / (n + tq[:, 0:1])
    ax2 = s_ref[1] / (n + tq[:, 1:2])
    ay1 = s_ref[2] / (n + tqy[:, 1:2])
    ay2 = s_ref[3] / (n + tqy[:, 0:1])
    qm_x1 = _dot(qx, matx_ref[0], ((1,), (0,)))
    qm_x2 = _dot(qx, maty_ref[0], ((1,), (0,)))
    qm_y1 = _dot(qy, maty_ref[0], ((1,), (0,)))
    qm_y2 = _dot(qy, matx_ref[0], ((1,), (0,)))
    vs = vs_ref[0]
    vys = vys_ref[0]
    fx_ref[0] = ax1 * (vs + qm_x1) + ax2 * (vs + qm_x2)
    fy_ref[0] = ay1 * (vys + qm_y1) + ay2 * (vys + qm_y2)


def _run(x, y, Wq, bq, Wk, bk, Wv, bv, Wqy, bqy, Wky, bky, Wvy, bvy,
         gamma, gamma_y, gamma_cx, gamma_cy, wx1, wx2, wy1, wy2,
         interpret=False):
    b, n, c = x.shape
    d = Wq.shape[0]
    bn1 = min(2048, n)
    bn2 = min(2048, n)
    nb1 = n // bn1
    nb2 = n // bn2

    row_spec = lambda bn: pl.BlockSpec((1, bn, c), lambda bi, i: (bi, i, 0))
    w_spec = lambda r, cc: pl.BlockSpec((r, cc), lambda bi, i: (0, 0))
    stat_spec = lambda r, cc: pl.BlockSpec((1, r, cc), lambda bi, i: (bi, 0, 0))
    f32 = jnp.float32

    import functools
    stats = pl.pallas_call(
        functools.partial(_stats_kernel, nb1),
        grid=(b, nb1),
        in_specs=[
            row_spec(bn1), row_spec(bn1),
            w_spec(d, c), w_spec(1, d), w_spec(d, c), w_spec(1, d),
            w_spec(c, c), w_spec(1, c),
            w_spec(d, c), w_spec(1, d), w_spec(d, c), w_spec(1, d),
            w_spec(c, c), w_spec(1, c),
        ],
        out_specs=[
            pl.BlockSpec((1, bn1, d), lambda bi, i: (bi, i, 0)),
            pl.BlockSpec((1, bn1, d), lambda bi, i: (bi, i, 0)),
            stat_spec(d, c), stat_spec(d, c),
            stat_spec(1, d), stat_spec(1, d),
            stat_spec(1, c), stat_spec(1, c),
        ],
        out_shape=[
            jax.ShapeDtypeStruct((b, n, d), f32),
            jax.ShapeDtypeStruct((b, n, d), f32),
            jax.ShapeDtypeStruct((b, d, c), f32),
            jax.ShapeDtypeStruct((b, d, c), f32),
            jax.ShapeDtypeStruct((b, 1, d), f32),
            jax.ShapeDtypeStruct((b, 1, d), f32),
            jax.ShapeDtypeStruct((b, 1, c), f32),
            jax.ShapeDtypeStruct((b, 1, c), f32),
        ],
        compiler_params=pltpu.CompilerParams(
            dimension_semantics=("parallel", "arbitrary")),
        name="linattn_stats",
        interpret=interpret,
    )(x, y,
      Wq, bq.reshape(1, d), Wk, bk.reshape(1, d), Wv, bv.reshape(1, c),
      Wqy, bqy.reshape(1, d), Wky, bky.reshape(1, d), Wvy, bvy.reshape(1, c))
    qx, qy, matx, maty, ks, kys, vs, vys = stats

    kp = jnp.transpose(jnp.concatenate([ks, kys], axis=1), (0, 2, 1))
    s = jnp.stack([gamma[0] * wx1, gamma_cx[0] * wx2,
                   gamma_y[0] * wy1, gamma_cy[0] * wy2]).astype(f32)

    fx, fy = pl.pallas_call(
        functools.partial(_out_kernel, n),
        grid=(b, nb2),
        in_specs=[
            pl.BlockSpec((1, bn2, d), lambda bi, i: (bi, i, 0)),
            pl.BlockSpec((1, bn2, d), lambda bi, i: (bi, i, 0)),
            stat_spec(d, c), stat_spec(d, c),
            pl.BlockSpec((1, d, 2), lambda bi, i: (bi, 0, 0)),
            stat_spec(1, c), stat_spec(1, c),
            pl.BlockSpec(memory_space=pltpu.SMEM),
        ],
        out_specs=[row_spec(bn2), row_spec(bn2)],
        out_shape=[
            jax.ShapeDtypeStruct((b, n, c), f32),
            jax.ShapeDtypeStruct((b, n, c), f32),
        ],
        compiler_params=pltpu.CompilerParams(
            dimension_semantics=("parallel", "arbitrary")),
        name="linattn_out",
        interpret=interpret,
    )(qx, qy, matx, maty, kp, vs, vys, s)
    return fx, fy


def kernel(x, y, Wq, bq, Wk, bk, Wv, bv, Wqy, bqy, Wky, bky, Wvy, bvy,
           gamma, gamma_y, gamma_cx, gamma_cy, wx1, wx2, wy1, wy2):
    return _run(x, y, Wq, bq, Wk, bk, Wv, bv, Wqy, bqy, Wky, bky, Wvy, bvy,
                gamma, gamma_y, gamma_cx, gamma_cy, wx1, wx2, wy1, wy2)


# speedup vs baseline: 2.7332x; 1.1795x over previous
import functools

import jax
import jax.numpy as jnp
from jax.experimental import pallas as pl
from jax.experimental.pallas import tpu as pltpu

EPS = 1e-6


def _dot(a, b, dims):
    return jax.lax.dot_general(a, b, (dims, ((), ())),
                               preferred_element_type=jnp.float32)


def _fused_kernel(nbatch, nb, bn,
                  x_ref, y_ref, wq_ref, bq_ref, wk_ref, bk_ref, wv_ref, bv_ref,
                  wqy_ref, bqy_ref, wky_ref, bky_ref, wvy_ref, bvy_ref, s_ref,
                  fx_ref, fy_ref,
                  qx_s, qy_s, matx_s, maty_s, ks_s, kys_s, vs_s, vys_s, kp_s):
    g = pl.program_id(0)
    j = pl.program_id(1)
    cur = jax.lax.rem(g, 2)
    prv = 1 - cur

    @pl.when(jnp.logical_and(g < nbatch, j == 0))
    def _():
        matx_s[cur] = jnp.zeros_like(matx_s[cur])
        maty_s[cur] = jnp.zeros_like(maty_s[cur])
        ks_s[cur] = jnp.zeros_like(ks_s[cur])
        kys_s[cur] = jnp.zeros_like(kys_s[cur])
        vs_s[cur] = jnp.zeros_like(vs_s[cur])
        vys_s[cur] = jnp.zeros_like(vys_s[cur])

    @pl.when(g < nbatch)
    def _phase_a():
        rows = pl.ds(j * bn, bn)

        def stream(t_ref, wq, bq, wk, bk, wv, bv, q_s, mat_s, ks, vs):
            t = t_ref[0]
            q = _dot(t, wq[...], ((1,), (1,))) + bq[...]
            q = q * jax.lax.rsqrt(jnp.sum(q * q, axis=1, keepdims=True))
            k = _dot(t, wk[...], ((1,), (1,))) + bk[...]
            k = k * jax.lax.rsqrt(jnp.sum(k * k, axis=1, keepdims=True))
            v = _dot(t, wv[...], ((1,), (1,))) + bv[...]
            q_s[cur, rows, :] = q
            mat_s[cur] += _dot(k, v, ((0,), (0,)))
            ks[cur] += jnp.sum(k, axis=0, keepdims=True)
            vs[cur] += jnp.sum(v, axis=0, keepdims=True)

        stream(x_ref, wq_ref, bq_ref, wk_ref, bk_ref, wv_ref, bv_ref,
               qx_s, matx_s, ks_s, vs_s)
        stream(y_ref, wqy_ref, bqy_ref, wky_ref, bky_ref, wvy_ref, bvy_ref,
               qy_s, maty_s, kys_s, vys_s)

    @pl.when(jnp.logical_and(g > 0, j == 0))
    def _():
        kp_s[:, 0:1] = jnp.transpose(ks_s[prv], (1, 0)) + EPS
        kp_s[:, 1:2] = jnp.transpose(kys_s[prv], (1, 0)) + EPS

    @pl.when(g > 0)
    def _phase_b():
        rows = pl.ds(j * bn, bn)
        qx = qx_s[prv, rows, :]
        qy = qy_s[prv, rows, :]
        tq = _dot(qx, kp_s[...], ((1,), (0,)))
        tqy = _dot(qy, kp_s[...], ((1,), (0,)))
        n = jnp.float32(nb * bn)
        ax1 = s_ref[0] / (n + tq[:, 0:1])
        ax2 = s_ref[1] / (n + tq[:, 1:2])
        ay1 = s_ref[2] / (n + tqy[:, 1:2])
        ay2 = s_ref[3] / (n + tqy[:, 0:1])
        qm_x1 = _dot(qx, matx_s[prv], ((1,), (0,)))
        qm_x2 = _dot(qx, maty_s[prv], ((1,), (0,)))
        qm_y1 = _dot(qy, maty_s[prv], ((1,), (0,)))
        qm_y2 = _dot(qy, matx_s[prv], ((1,), (0,)))
        vs = vs_s[prv][0:1]
        vys = vys_s[prv][0:1]
        fx_ref[0] = ax1 * (vs + qm_x1) + ax2 * (vs + qm_x2)
        fy_ref[0] = ay1 * (vys + qm_y1) + ay2 * (vys + qm_y2)


def _run(x, y, Wq, bq, Wk, bk, Wv, bv, Wqy, bqy, Wky, bky, Wvy, bvy,
         gamma, gamma_y, gamma_cx, gamma_cy, wx1, wx2, wy1, wy2,
         interpret=False):
    b, n, c = x.shape
    d = Wq.shape[0]
    bn = min(2048, n)
    nb = n // bn
    f32 = jnp.float32

    in_map = lambda g, j: (jnp.where(g < b, g, b - 1),
                           jnp.where(g < b, j, nb - 1), 0)
    out_map = lambda g, j: (jnp.maximum(g - 1, 0),
                            jnp.where(g > 0, j, 0), 0)
    row_in = pl.BlockSpec((1, bn, c), in_map)
    row_out = pl.BlockSpec((1, bn, c), out_map)
    w_spec = lambda r, cc: pl.BlockSpec((r, cc), lambda g, j: (0, 0))

    s = jnp.stack([gamma[0] * wx1, gamma_cx[0] * wx2,
                   gamma_y[0] * wy1, gamma_cy[0] * wy2]).astype(f32)

    fx, fy = pl.pallas_call(
        functools.partial(_fused_kernel, b, nb, bn),
        grid=(b + 1, nb),
        in_specs=[
            row_in, row_in,
            w_spec(d, c), w_spec(1, d), w_spec(d, c), w_spec(1, d),
            w_spec(c, c), w_spec(1, c),
            w_spec(d, c), w_spec(1, d), w_spec(d, c), w_spec(1, d),
            w_spec(c, c), w_spec(1, c),
            pl.BlockSpec(memory_space=pltpu.SMEM),
        ],
        out_specs=[row_out, row_out],
        out_shape=[
            jax.ShapeDtypeStruct((b, n, c), f32),
            jax.ShapeDtypeStruct((b, n, c), f32),
        ],
        scratch_shapes=[
            pltpu.VMEM((2, n, d), f32),
            pltpu.VMEM((2, n, d), f32),
            pltpu.VMEM((2, d, c), f32),
            pltpu.VMEM((2, d, c), f32),
            pltpu.VMEM((2, 1, d), f32),
            pltpu.VMEM((2, 1, d), f32),
            pltpu.VMEM((2, 1, c), f32),
            pltpu.VMEM((2, 1, c), f32),
            pltpu.VMEM((d, 2), f32),
        ],
        compiler_params=pltpu.CompilerParams(
            dimension_semantics=("arbitrary", "arbitrary"),
            vmem_limit_bytes=56 * 1024 * 1024),
        name="linattn_fused",
        interpret=interpret,
    )(x, y,
      Wq, bq.reshape(1, d), Wk, bk.reshape(1, d), Wv, bv.reshape(1, c),
      Wqy, bqy.reshape(1, d), Wky, bky.reshape(1, d), Wvy, bvy.reshape(1, c),
      s)
    return fx, fy


def kernel(x, y, Wq, bq, Wk, bk, Wv, bv, Wqy, bqy, Wky, bky, Wvy, bvy,
           gamma, gamma_y, gamma_cx, gamma_cy, wx1, wx2, wy1, wy2):
    return _run(x, y, Wq, bq, Wk, bk, Wv, bv, Wqy, bqy, Wky, bky, Wvy, bvy,
                gamma, gamma_y, gamma_cx, gamma_cy, wx1, wx2, wy1, wy2)
